# baseline (speedup 1.0000x reference)
import functools

import jax
import jax.numpy as jnp
from jax import lax
from jax.experimental import pallas as pl
from jax.experimental.pallas import tpu as pltpu
from jax.experimental.pallas import tpu_sc as plsc

_N = 10000
_E = 320000
_D = 128
_NC = 2
_NS = 16
_NW = _NC * _NS
_CHUNK = 64
_NCHUNK = 160
_WIN = 40
_EW = _E // _NW
_EWP = _NCHUNK * _CHUNK
_CNTW = 128
_NP = 10112
_RPT = _NP // _NS


def _mesh():
    return plsc.VectorSubcoreMesh(
        core_axis_name="c", subcore_axis_name="s",
        num_cores=_NC, num_subcores=_NS)


def _worker_ids():
    c = lax.axis_index("c")
    s = lax.axis_index("s")
    return c, s, c * _NS + s


def _sc_agg_body(h_hbm, src_hbm, dst_hbm, z128_hbm, out_hbm,
                 src_v, dst_v, rows0_v, rows1_v, rows2_v, rows3_v, acc_sh,
                 semg0, semg1, semg2, semg3, sems0, sems1, sems2, sems3):
    c, s, wid = _worker_ids()
    row0 = s * _RPT
    pltpu.sync_copy(z128_hbm, acc_sh.at[pl.ds(row0, _RPT)])
    plsc.subcore_barrier()

    def gather(j, buf, sem):
        pltpu.async_copy(h_hbm.at[src_v.at[j]], buf, sem)

    def scatter(j, buf, sem):
        pltpu.async_copy(buf, acc_sh.at[dst_v.at[j]], sem, add=True)

    def drain_gather(j, buf, sem):
        pltpu.make_async_copy(h_hbm.at[src_v.at[j]], buf, sem).wait()

    def drain_scatter(j, buf, sem):
        pltpu.make_async_copy(buf, acc_sh.at[dst_v.at[j]], sem).wait()

    def phase(base):
        pltpu.sync_copy(src_hbm.at[wid, pl.ds(base, _WIN)], src_v)
        pltpu.sync_copy(dst_hbm.at[wid, pl.ds(base, _WIN)], dst_v)
        bufs = (rows0_v, rows1_v, rows2_v, rows3_v)
        gsems = (semg0, semg1, semg2, semg3)
        ssems = (sems0, sems1, sems2, sems3)
        for b in range(3):
            gather(b, bufs[b], gsems[b])

        def step(j, b, first=False):
            drain_gather(j, bufs[b], gsems[b])
            scatter(j, bufs[b], ssems[b])
            bp = (b + 3) % 4
            if not first:
                drain_scatter(j - 1, bufs[bp], ssems[bp])
            gather(jnp.minimum(j + 3, _WIN - 1), bufs[bp], gsems[bp])

        step(0, 0, first=True)
        for k in (1, 2, 3):
            step(k, k)

        def body(i, carry):
            j0 = 4 * i + 4
            for k in range(4):
                step(j0 + k, k)
            return carry

        lax.fori_loop(0, (_WIN - 4) // 4, body, 0)
        drain_scatter(_WIN - 1, bufs[3], ssems[3])
        for k in range(3):
            drain_gather(_WIN - 1, bufs[k], gsems[k])

    for p in range(_NCHUNK // _WIN):
        phase(p * _WIN)

    plsc.subcore_barrier()
    pltpu.sync_copy(acc_sh.at[pl.ds(row0, _RPT)],
                    out_hbm.at[c, pl.ds(row0, _RPT)])


def _sc_cnt_body(dst_hbm, z128_hbm, ones_hbm, cnt_out_hbm,
                 dst_v, ones_v, cnt_sh):
    c, s, wid = _worker_ids()
    row0 = s * _RPT
    pltpu.sync_copy(z128_hbm, cnt_sh.at[pl.ds(row0, _RPT)])
    pltpu.sync_copy(ones_hbm, ones_v)
    pltpu.sync_copy(dst_hbm.at[wid], dst_v)
    plsc.subcore_barrier()

    def body(j, carry):
        pltpu.sync_copy(ones_v, cnt_sh.at[dst_v.at[j]], add=True)
        return carry

    lax.fori_loop(0, _NCHUNK, body, 0)
    plsc.subcore_barrier()
    pltpu.sync_copy(cnt_sh.at[pl.ds(row0, _RPT)],
                    cnt_out_hbm.at[c, pl.ds(row0, _RPT)])


def _make_agg(interpret=False):
    return pl.kernel(
        _sc_agg_body,
        out_type=jax.ShapeDtypeStruct((_NC, _NP, _D), jnp.float32),
        mesh=_mesh(),
        scratch_types=[
            pltpu.VMEM((_WIN, _CHUNK), jnp.int32),
            pltpu.VMEM((_WIN, _CHUNK), jnp.int32),
            pltpu.VMEM((_CHUNK, _D), jnp.float32),
            pltpu.VMEM((_CHUNK, _D), jnp.float32),
            pltpu.VMEM((_CHUNK, _D), jnp.float32),
            pltpu.VMEM((_CHUNK, _D), jnp.float32),
            pltpu.VMEM_SHARED((_NP, _D), jnp.float32),
        ] + [pltpu.SemaphoreType.DMA] * 8,
        interpret=interpret,
    )


def _make_cnt(interpret=False):
    return pl.kernel(
        _sc_cnt_body,
        out_type=jax.ShapeDtypeStruct((_NC, _NP, _CNTW), jnp.float32),
        mesh=_mesh(),
        scratch_types=[
            pltpu.VMEM((_NCHUNK, _CHUNK), jnp.int32),
            pltpu.VMEM((_CHUNK, _CNTW), jnp.float32),
            pltpu.VMEM_SHARED((_NP, _CNTW), jnp.float32),
        ],
        interpret=interpret,
    )


def _dense_body(relu, p_ref, c_ref, h_ref, wl_ref, wr_ref, bl_ref, o_ref):
    ssum = p_ref[0] + p_ref[1]
    cnt = c_ref[0, :, 0:1] + c_ref[1, :, 0:1]
    mean = ssum / jnp.maximum(cnt, 1.0)
    acc = lax.dot_general(mean, wl_ref[...], (((1,), (1,)), ((), ())),
                          preferred_element_type=jnp.float32)
    acc = acc + lax.dot_general(h_ref[...], wr_ref[...], (((1,), (1,)), ((), ())),
                                preferred_element_type=jnp.float32)
    acc = acc + bl_ref[...]
    o_ref[...] = jnp.maximum(acc, 0.0) if relu else acc


def _dense(part, cnt, h, Wl, bl, Wr, relu, interpret=False):
    bm = 632
    return pl.pallas_call(
        functools.partial(_dense_body, relu),
        grid=(_NP // bm,),
        in_specs=[
            pl.BlockSpec((_NC, bm, _D), lambda i: (0, i, 0)),
            pl.BlockSpec((_NC, bm, _CNTW), lambda i: (0, i, 0)),
            pl.BlockSpec((bm, _D), lambda i: (i, 0)),
            pl.BlockSpec((_D, _D), lambda i: (0, 0)),
            pl.BlockSpec((_D, _D), lambda i: (0, 0)),
            pl.BlockSpec((1, _D), lambda i: (0, 0)),
        ],
        out_specs=pl.BlockSpec((bm, _D), lambda i: (i, 0)),
        out_shape=jax.ShapeDtypeStruct((_NP, _D), jnp.float32),
        interpret=interpret,
    )(part, cnt, h, Wl, Wr, bl.reshape(1, _D))


def kernel(x, edge_index, Wl1, bl1, Wr1, Wl2, bl2, Wr2, Wl3, bl3, Wr3):
    xp = jnp.pad(x, ((0, _NP - _N), (0, 0)))
    padn = _EWP - _EW
    pad_dst = (_N + 3 * jnp.arange(_NW, dtype=jnp.int32)[:, None]
               + (jnp.arange(padn, dtype=jnp.int32)[None, :] % 3))
    pad_src = jnp.broadcast_to(
        (37 * jnp.arange(padn, dtype=jnp.int32)) % _N, (_NW, padn))
    src = jnp.concatenate(
        [edge_index[0].astype(jnp.int32).reshape(_NW, _EW), pad_src], axis=1,
    ).reshape(_NW, _NCHUNK, _CHUNK)
    dst = jnp.concatenate(
        [edge_index[1].astype(jnp.int32).reshape(_NW, _EW), pad_dst], axis=1,
    ).reshape(_NW, _NCHUNK, _CHUNK)
    z128 = jnp.zeros((_RPT, _D), jnp.float32)
    ones = jnp.ones((_CHUNK, _CNTW), jnp.float32)

    agg = _make_agg()
    cnt = _make_cnt()(dst, z128, ones)

    part1 = agg(xp, src, dst, z128)
    h1 = _dense(part1, cnt, xp, Wl1, bl1, Wr1, True)
    part2 = agg(h1, src, dst, z128)
    h2 = _dense(part2, cnt, h1, Wl2, bl2, Wr2, True)
    part3 = agg(h2, src, dst, z128)
    out = _dense(part3, cnt, h2, Wl3, bl3, Wr3, False)
    return out[:_N]


# speedup vs baseline: 1.0612x; 1.0612x over previous
import functools

import jax
import jax.numpy as jnp
from jax import lax
from jax.experimental import pallas as pl
from jax.experimental.pallas import tpu as pltpu
from jax.experimental.pallas import tpu_sc as plsc

_N = 10000
_E = 320000
_D = 128
_NC = 2
_NS = 16
_NW = _NC * _NS
_CHUNK = 128
_NCHUNK = 80
_WIN = 40
_EW = _E // _NW
_EWP = _NCHUNK * _CHUNK
_CNTW = 128
_NP = 10112
_RPT = _NP // _NS


def _mesh():
    return plsc.VectorSubcoreMesh(
        core_axis_name="c", subcore_axis_name="s",
        num_cores=_NC, num_subcores=_NS)


def _worker_ids():
    c = lax.axis_index("c")
    s = lax.axis_index("s")
    return c, s, c * _NS + s


def _sc_agg_body(h_hbm, src_hbm, dst_hbm, z128_hbm, out_hbm,
                 src_v, dst_v, rows0_v, rows1_v, acc_sh,
                 semg0, semg1, sems0, sems1):
    c, s, wid = _worker_ids()
    row0 = s * _RPT
    pltpu.sync_copy(z128_hbm, acc_sh.at[pl.ds(row0, _RPT)])
    plsc.subcore_barrier()

    def gather(j, buf, sem):
        pltpu.async_copy(h_hbm.at[src_v.at[j]], buf, sem)

    def scatter(j, buf, sem):
        pltpu.async_copy(buf, acc_sh.at[dst_v.at[j]], sem, add=True)

    def drain_gather(j, buf, sem):
        pltpu.make_async_copy(h_hbm.at[src_v.at[j]], buf, sem).wait()

    def drain_scatter(j, buf, sem):
        pltpu.make_async_copy(buf, acc_sh.at[dst_v.at[j]], sem).wait()

    def phase(base):
        pltpu.sync_copy(src_hbm.at[wid, pl.ds(base, _WIN)], src_v)
        pltpu.sync_copy(dst_hbm.at[wid, pl.ds(base, _WIN)], dst_v)
        bufs = (rows0_v, rows1_v)
        gsems = (semg0, semg1)
        ssems = (sems0, sems1)
        gather(0, bufs[0], gsems[0])

        def step(j, b, first=False):
            bp = (b + 1) % 2
            if not first:
                drain_scatter(j - 1, bufs[bp], ssems[bp])
            gather(jnp.minimum(j + 1, _WIN - 1), bufs[bp], gsems[bp])
            drain_gather(j, bufs[b], gsems[b])
            scatter(j, bufs[b], ssems[b])

        step(0, 0, first=True)
        step(1, 1)

        def body(i, carry):
            j0 = 2 * i + 2
            step(j0, 0)
            step(j0 + 1, 1)
            return carry

        lax.fori_loop(0, (_WIN - 2) // 2, body, 0)
        drain_scatter(_WIN - 1, bufs[1], ssems[1])
        drain_gather(_WIN - 1, bufs[0], gsems[0])

    for p in range(_NCHUNK // _WIN):
        phase(p * _WIN)

    plsc.subcore_barrier()
    pltpu.sync_copy(acc_sh.at[pl.ds(row0, _RPT)],
                    out_hbm.at[c, pl.ds(row0, _RPT)])


def _sc_cnt_body(dst_hbm, z128_hbm, ones_hbm, cnt_out_hbm,
                 dst_v, ones_v, cnt_sh):
    c, s, wid = _worker_ids()
    row0 = s * _RPT
    pltpu.sync_copy(z128_hbm, cnt_sh.at[pl.ds(row0, _RPT)])
    pltpu.sync_copy(ones_hbm, ones_v)
    pltpu.sync_copy(dst_hbm.at[wid], dst_v)
    plsc.subcore_barrier()

    def body(j, carry):
        pltpu.sync_copy(ones_v, cnt_sh.at[dst_v.at[j]], add=True)
        return carry

    lax.fori_loop(0, _NCHUNK, body, 0)
    plsc.subcore_barrier()
    pltpu.sync_copy(cnt_sh.at[pl.ds(row0, _RPT)],
                    cnt_out_hbm.at[c, pl.ds(row0, _RPT)])


def _make_agg(interpret=False):
    return pl.kernel(
        _sc_agg_body,
        out_type=jax.ShapeDtypeStruct((_NC, _NP, _D), jnp.float32),
        mesh=_mesh(),
        scratch_types=[
            pltpu.VMEM((_WIN, _CHUNK), jnp.int32),
            pltpu.VMEM((_WIN, _CHUNK), jnp.int32),
            pltpu.VMEM((_CHUNK, _D), jnp.float32),
            pltpu.VMEM((_CHUNK, _D), jnp.float32),
            pltpu.VMEM_SHARED((_NP, _D), jnp.float32),
        ] + [pltpu.SemaphoreType.DMA] * 4,
        interpret=interpret,
    )


def _make_cnt(interpret=False):
    return pl.kernel(
        _sc_cnt_body,
        out_type=jax.ShapeDtypeStruct((_NC, _NP, _CNTW), jnp.float32),
        mesh=_mesh(),
        scratch_types=[
            pltpu.VMEM((_NCHUNK, _CHUNK), jnp.int32),
            pltpu.VMEM((_CHUNK, _CNTW), jnp.float32),
            pltpu.VMEM_SHARED((_NP, _CNTW), jnp.float32),
        ],
        interpret=interpret,
    )


def _dense_body(relu, p_ref, c_ref, h_ref, wl_ref, wr_ref, bl_ref, o_ref):
    ssum = p_ref[0] + p_ref[1]
    cnt = c_ref[0, :, 0:1] + c_ref[1, :, 0:1]
    mean = ssum / jnp.maximum(cnt, 1.0)
    acc = lax.dot_general(mean, wl_ref[...], (((1,), (1,)), ((), ())),
                          preferred_element_type=jnp.float32)
    acc = acc + lax.dot_general(h_ref[...], wr_ref[...], (((1,), (1,)), ((), ())),
                                preferred_element_type=jnp.float32)
    acc = acc + bl_ref[...]
    o_ref[...] = jnp.maximum(acc, 0.0) if relu else acc


def _dense(part, cnt, h, Wl, bl, Wr, relu, interpret=False):
    bm = 632
    return pl.pallas_call(
        functools.partial(_dense_body, relu),
        grid=(_NP // bm,),
        in_specs=[
            pl.BlockSpec((_NC, bm, _D), lambda i: (0, i, 0)),
            pl.BlockSpec((_NC, bm, 8), lambda i: (0, i, 0)),
            pl.BlockSpec((bm, _D), lambda i: (i, 0)),
            pl.BlockSpec((_D, _D), lambda i: (0, 0)),
            pl.BlockSpec((_D, _D), lambda i: (0, 0)),
            pl.BlockSpec((1, _D), lambda i: (0, 0)),
        ],
        out_specs=pl.BlockSpec((bm, _D), lambda i: (i, 0)),
        out_shape=jax.ShapeDtypeStruct((_NP, _D), jnp.float32),
        interpret=interpret,
    )(part, cnt, h, Wl, Wr, bl.reshape(1, _D))


def kernel(x, edge_index, Wl1, bl1, Wr1, Wl2, bl2, Wr2, Wl3, bl3, Wr3):
    xp = jnp.pad(x, ((0, _NP - _N), (0, 0)))
    padn = _EWP - _EW
    pad_dst = (_N + 3 * jnp.arange(_NW, dtype=jnp.int32)[:, None]
               + (jnp.arange(padn, dtype=jnp.int32)[None, :] % 3))
    pad_src = jnp.broadcast_to(
        (37 * jnp.arange(padn, dtype=jnp.int32)) % _N, (_NW, padn))
    src = jnp.concatenate(
        [edge_index[0].astype(jnp.int32).reshape(_NW, _EW), pad_src], axis=1,
    ).reshape(_NW, _NCHUNK, _CHUNK)
    dst = jnp.concatenate(
        [edge_index[1].astype(jnp.int32).reshape(_NW, _EW), pad_dst], axis=1,
    ).reshape(_NW, _NCHUNK, _CHUNK)
    z128 = jnp.zeros((_RPT, _D), jnp.float32)
    ones = jnp.ones((_CHUNK, _CNTW), jnp.float32)

    agg = _make_agg()
    cnt = _make_cnt()(dst, z128, ones)[:, :, :8]

    part1 = agg(xp, src, dst, z128)
    h1 = _dense(part1, cnt, xp, Wl1, bl1, Wr1, True)
    part2 = agg(h1, src, dst, z128)
    h2 = _dense(part2, cnt, h1, Wl2, bl2, Wr2, True)
    part3 = agg(h2, src, dst, z128)
    out = _dense(part3, cnt, h2, Wl3, bl3, Wr3, False)
    return out[:_N]


# speedup vs baseline: 1.0614x; 1.0002x over previous
import functools

import jax
import jax.numpy as jnp
from jax import lax
from jax.experimental import pallas as pl
from jax.experimental.pallas import tpu as pltpu
from jax.experimental.pallas import tpu_sc as plsc

_N = 10000
_E = 320000
_D = 128
_NC = 2
_NS = 16
_NW = _NC * _NS
_CHUNK = 128
_NCHUNK = 80
_WIN = 40
_EW = _E // _NW
_EWP = _NCHUNK * _CHUNK
_CNTW = 128
_NP = 10112
_RPT = _NP // _NS


def _mesh():
    return plsc.VectorSubcoreMesh(
        core_axis_name="c", subcore_axis_name="s",
        num_cores=_NC, num_subcores=_NS)


def _worker_ids():
    c = lax.axis_index("c")
    s = lax.axis_index("s")
    return c, s, c * _NS + s


def _sc_agg_body(h_hbm, src_hbm, dst_hbm, z128_hbm, out_hbm,
                 src_v, dst_v, rows0_v, rows1_v, acc_sh,
                 semg0, semg1, sems0, sems1):
    c, s, wid = _worker_ids()
    row0 = s * _RPT
    pltpu.sync_copy(z128_hbm, acc_sh.at[pl.ds(row0, _RPT)])
    plsc.subcore_barrier()

    def gather(j, buf, sem):
        pltpu.async_copy(h_hbm.at[src_v.at[j]], buf, sem)

    def scatter(j, buf, sem):
        pltpu.async_copy(buf, acc_sh.at[dst_v.at[j]], sem, add=True)

    def drain_gather(j, buf, sem):
        pltpu.make_async_copy(h_hbm.at[src_v.at[j]], buf, sem).wait()

    def drain_scatter(j, buf, sem):
        pltpu.make_async_copy(buf, acc_sh.at[dst_v.at[j]], sem).wait()

    def phase(base):
        pltpu.sync_copy(src_hbm.at[wid, pl.ds(base, _WIN)], src_v)
        pltpu.sync_copy(dst_hbm.at[wid, pl.ds(base, _WIN)], dst_v)
        bufs = (rows0_v, rows1_v)
        gsems = (semg0, semg1)
        ssems = (sems0, sems1)
        gather(0, bufs[0], gsems[0])

        def step(j, b, first=False):
            bp = (b + 1) % 2
            if not first:
                drain_scatter(j - 1, bufs[bp], ssems[bp])
            gather(jnp.minimum(j + 1, _WIN - 1), bufs[bp], gsems[bp])
            drain_gather(j, bufs[b], gsems[b])
            scatter(j, bufs[b], ssems[b])

        step(0, 0, first=True)
        step(1, 1)

        def body(i, carry):
            j0 = 2 * i + 2
            step(j0, 0)
            step(j0 + 1, 1)
            return carry

        lax.fori_loop(0, (_WIN - 2) // 2, body, 0)
        drain_scatter(_WIN - 1, bufs[1], ssems[1])
        drain_gather(_WIN - 1, bufs[0], gsems[0])

    for p in range(_NCHUNK // _WIN):
        phase(p * _WIN)

    plsc.subcore_barrier()
    pltpu.sync_copy(acc_sh.at[pl.ds(row0, _RPT)],
                    out_hbm.at[c, pl.ds(row0, _RPT)])


def _sc_cnt_body(dst_hbm, z128_hbm, ones_hbm, cnt_out_hbm,
                 dst_v, ones_v, cnt_sh, sem):
    c, s, wid = _worker_ids()
    row0 = s * _RPT
    pltpu.sync_copy(z128_hbm, cnt_sh.at[pl.ds(row0, _RPT)])
    pltpu.sync_copy(ones_hbm, ones_v)
    pltpu.sync_copy(dst_hbm.at[wid], dst_v)
    plsc.subcore_barrier()

    def issue(j):
        pltpu.async_copy(ones_v, cnt_sh.at[dst_v.at[j]], sem, add=True)

    def drain(j):
        pltpu.make_async_copy(ones_v, cnt_sh.at[dst_v.at[j]], sem).wait()

    for k in range(4):
        issue(k)

    def body(j, carry):
        drain(j - 4)
        issue(j)
        return carry

    lax.fori_loop(4, _NCHUNK, body, 0)
    for k in range(_NCHUNK - 4, _NCHUNK):
        drain(k)
    plsc.subcore_barrier()
    pltpu.sync_copy(cnt_sh.at[pl.ds(row0, _RPT)],
                    cnt_out_hbm.at[c, pl.ds(row0, _RPT)])


def _make_agg(interpret=False):
    return pl.kernel(
        _sc_agg_body,
        out_type=jax.ShapeDtypeStruct((_NC, _NP, _D), jnp.float32),
        mesh=_mesh(),
        scratch_types=[
            pltpu.VMEM((_WIN, _CHUNK), jnp.int32),
            pltpu.VMEM((_WIN, _CHUNK), jnp.int32),
            pltpu.VMEM((_CHUNK, _D), jnp.float32),
            pltpu.VMEM((_CHUNK, _D), jnp.float32),
            pltpu.VMEM_SHARED((_NP, _D), jnp.float32),
        ] + [pltpu.SemaphoreType.DMA] * 4,
        interpret=interpret,
    )


def _make_cnt(interpret=False):
    return pl.kernel(
        _sc_cnt_body,
        out_type=jax.ShapeDtypeStruct((_NC, _NP, _CNTW), jnp.float32),
        mesh=_mesh(),
        scratch_types=[
            pltpu.VMEM((_NCHUNK, _CHUNK), jnp.int32),
            pltpu.VMEM((_CHUNK, _CNTW), jnp.float32),
            pltpu.VMEM_SHARED((_NP, _CNTW), jnp.float32),
            pltpu.SemaphoreType.DMA,
        ],
        interpret=interpret,
    )


def _dense_body(relu, p_ref, c_ref, h_ref, wl_ref, wr_ref, bl_ref, o_ref):
    ssum = p_ref[0] + p_ref[1]
    cnt = c_ref[0, :, 0:1] + c_ref[1, :, 0:1]
    mean = ssum / jnp.maximum(cnt, 1.0)
    acc = lax.dot_general(mean, wl_ref[...], (((1,), (1,)), ((), ())),
                          preferred_element_type=jnp.float32)
    acc = acc + lax.dot_general(h_ref[...], wr_ref[...], (((1,), (1,)), ((), ())),
                                preferred_element_type=jnp.float32)
    acc = acc + bl_ref[...]
    o_ref[...] = jnp.maximum(acc, 0.0) if relu else acc


def _dense(part, cnt, h, Wl, bl, Wr, relu, interpret=False):
    bm = 632
    return pl.pallas_call(
        functools.partial(_dense_body, relu),
        grid=(_NP // bm,),
        in_specs=[
            pl.BlockSpec((_NC, bm, _D), lambda i: (0, i, 0)),
            pl.BlockSpec((_NC, bm, 8), lambda i: (0, i, 0)),
            pl.BlockSpec((bm, _D), lambda i: (i, 0)),
            pl.BlockSpec((_D, _D), lambda i: (0, 0)),
            pl.BlockSpec((_D, _D), lambda i: (0, 0)),
            pl.BlockSpec((1, _D), lambda i: (0, 0)),
        ],
        out_specs=pl.BlockSpec((bm, _D), lambda i: (i, 0)),
        out_shape=jax.ShapeDtypeStruct((_NP, _D), jnp.float32),
        interpret=interpret,
    )(part, cnt, h, Wl, Wr, bl.reshape(1, _D))


def kernel(x, edge_index, Wl1, bl1, Wr1, Wl2, bl2, Wr2, Wl3, bl3, Wr3):
    xp = jnp.pad(x, ((0, _NP - _N), (0, 0)))
    padn = _EWP - _EW
    pad_dst = (_N + 3 * jnp.arange(_NW, dtype=jnp.int32)[:, None]
               + (jnp.arange(padn, dtype=jnp.int32)[None, :] % 3))
    pad_src = jnp.broadcast_to(
        (37 * jnp.arange(padn, dtype=jnp.int32)) % _N, (_NW, padn))
    src = jnp.concatenate(
        [edge_index[0].astype(jnp.int32).reshape(_NW, _EW), pad_src], axis=1,
    ).reshape(_NW, _NCHUNK, _CHUNK)
    dst = jnp.concatenate(
        [edge_index[1].astype(jnp.int32).reshape(_NW, _EW), pad_dst], axis=1,
    ).reshape(_NW, _NCHUNK, _CHUNK)
    z128 = jnp.zeros((_RPT, _D), jnp.float32)
    ones = jnp.ones((_CHUNK, _CNTW), jnp.float32)

    agg = _make_agg()
    cnt = _make_cnt()(dst, z128, ones)[:, :, :8]

    part1 = agg(xp, src, dst, z128)
    h1 = _dense(part1, cnt, xp, Wl1, bl1, Wr1, True)
    part2 = agg(h1, src, dst, z128)
    h2 = _dense(part2, cnt, h1, Wl2, bl2, Wr2, True)
    part3 = agg(h2, src, dst, z128)
    out = _dense(part3, cnt, h2, Wl3, bl3, Wr3, False)
    return out[:_N]
